# Initial kernel scaffold; baseline (speedup 1.0000x reference)
import jax
import jax.numpy as jnp
from jax.experimental import pallas as pl


def kernel(x, params):
    raise NotImplementedError("write your pallas kernel here")


# speedup vs baseline: 12.0888x; 12.0888x over previous
import functools

import jax
import jax.numpy as jnp
from jax import lax
from jax.experimental import pallas as pl
from jax.experimental.pallas import tpu as pltpu
from jax.experimental.pallas import tpu_sc as plsc

N = 2048
K = 16
BLK = 256
NBLK = N // BLK


def _bf(a):
    return a.astype(jnp.bfloat16)


def _topk_body(feat_ref, sq_ref, sqr_ref, idx_ref):
    b = pl.program_id(0)
    feat = feat_ref[...]
    sq = sq_ref[...]
    ctr = feat_ref[pl.ds(b * BLK, BLK), :]
    mm = lax.dot_general(_bf(feat), _bf(ctr), (((1,), (1,)), ((), ())),
                         preferred_element_type=jnp.float32)
    sqi_row = sqr_ref[:, pl.ds(b * BLK, BLK)]
    dmat = (sqi_row - 2.0 * mm) + sq
    jj = lax.broadcasted_iota(jnp.int32, (N, BLK), 0)
    ii = b * BLK + lax.broadcasted_iota(jnp.int32, (N, BLK), 1)
    dmat = jnp.where(jj == ii, jnp.inf, dmat)
    rows = []
    for _ in range(K):
        m = jnp.min(dmat, axis=0, keepdims=True)
        cand = jnp.where(dmat == m, jj, N)
        sel = jnp.min(cand, axis=0, keepdims=True)
        rows.append(sel)
        dmat = jnp.where(jj == sel, jnp.inf, dmat)
    idx_ref[...] = jnp.concatenate(rows, axis=0)


def _topk(feat, sq):
    c = feat.shape[1]
    return pl.pallas_call(
        _topk_body,
        grid=(NBLK,),
        in_specs=[pl.BlockSpec((N, c), lambda b: (0, 0)),
                  pl.BlockSpec((N, 1), lambda b: (0, 0)),
                  pl.BlockSpec((1, N), lambda b: (0, 0))],
        out_specs=pl.BlockSpec((K, BLK), lambda b: (0, b)),
        out_shape=jax.ShapeDtypeStruct((K, N), jnp.int32),
    )(feat, sq[:, None], sq[None, :])


def _gather_rows(table, idx_flat):
    cp = table.shape[1]
    nw = 32
    b_per_w = (K * N) // nw
    ch = 256
    mesh = plsc.VectorSubcoreMesh(core_axis_name="c", subcore_axis_name="s")

    @functools.partial(
        pl.kernel, mesh=mesh,
        compiler_params=pltpu.CompilerParams(use_tc_tiling_on_sc=False),
        out_type=jax.ShapeDtypeStruct((K * N, cp), jnp.float32),
        scratch_types=[
            pltpu.VMEM((ch,), jnp.int32),
            pltpu.VMEM((ch, cp), jnp.float32),
            pltpu.SemaphoreType.DMA,
        ],
    )
    def gk(table_hbm, idx_hbm, out_hbm, idx_v, rows_v, sem):
        wid = lax.axis_index("s") * 2 + lax.axis_index("c")
        base = wid * b_per_w

        def step(ci, carry):
            off = base + ci * ch
            pltpu.sync_copy(idx_hbm.at[pl.ds(off, ch)], idx_v)
            pltpu.async_copy(table_hbm.at[idx_v], rows_v, sem).wait()
            pltpu.sync_copy(rows_v, out_hbm.at[pl.ds(off, ch)])
            return carry

        lax.fori_loop(0, b_per_w // ch, step, 0)

    return gk(table, idx_flat)


def _mlp_body(nb_ref, feat_ref, w1_ref, b1_ref, w2_ref, b2_ref, w3_ref,
              b3_ref, out_ref, *, c, h):
    ctr = feat_ref[...]
    nbb = nb_ref[...]
    if nbb.shape[-1] != c:
        nbb = nbb[:, :, :c]
    ctrb = jnp.broadcast_to(ctr[None, :, :], (K, BLK, c))
    h0 = jnp.concatenate([ctrb, nbb - ctrb], axis=-1)
    hh = h0.reshape(K * BLK, 2 * c)
    hh = jnp.dot(_bf(hh), _bf(w1_ref[...]),
                 preferred_element_type=jnp.float32) + b1_ref[...]
    hh = jnp.maximum(hh, 0.0)
    hh = jnp.dot(_bf(hh), _bf(w2_ref[...]),
                 preferred_element_type=jnp.float32) + b2_ref[...]
    hh = jnp.maximum(hh, 0.0)
    hh = jnp.dot(_bf(hh), _bf(w3_ref[...]),
                 preferred_element_type=jnp.float32) + b3_ref[...]
    h3 = hh.reshape(K, BLK, h)
    acc = h3[0]
    for k in range(1, K):
        acc = jnp.maximum(acc, h3[k])
    out_ref[...] = acc


def _mlp(nb3, feat, ws, bs):
    c = feat.shape[1]
    cp = nb3.shape[2]
    h = ws[2].shape[1]
    return pl.pallas_call(
        functools.partial(_mlp_body, c=c, h=h),
        grid=(NBLK,),
        in_specs=[
            pl.BlockSpec((K, BLK, cp), lambda b: (0, b, 0)),
            pl.BlockSpec((BLK, c), lambda b: (b, 0)),
            pl.BlockSpec(ws[0].shape, lambda b: (0, 0)),
            pl.BlockSpec(bs[0].shape, lambda b: (0,)),
            pl.BlockSpec(ws[1].shape, lambda b: (0, 0)),
            pl.BlockSpec(bs[1].shape, lambda b: (0,)),
            pl.BlockSpec(ws[2].shape, lambda b: (0, 0)),
            pl.BlockSpec(bs[2].shape, lambda b: (0,)),
        ],
        out_specs=pl.BlockSpec((BLK, h), lambda b: (b, 0)),
        out_shape=jax.ShapeDtypeStruct((N, h), jnp.float32),
    )(nb3, feat, ws[0], bs[0], ws[1], bs[1], ws[2], bs[2])


def kernel(x, params):
    feat = x
    for ws, bs in params:
        c = feat.shape[1]
        sq = jnp.sum(feat * feat, axis=1)
        idx = _topk(feat, sq)
        if c % 16:
            table = jnp.pad(feat, ((0, 0), (0, 16 - c)))
        else:
            table = feat
        nb = _gather_rows(table, idx.reshape(-1))
        nb3 = nb.reshape(K, N, table.shape[1])
        feat = _mlp(nb3, feat, ws, bs)
    return feat


# speedup vs baseline: 12.2094x; 1.0100x over previous
import functools

import jax
import jax.numpy as jnp
from jax import lax
from jax.experimental import pallas as pl
from jax.experimental.pallas import tpu as pltpu
from jax.experimental.pallas import tpu_sc as plsc

N = 2048
K = 16
BLK = 256
NBLK = N // BLK


def _bf(a):
    return a.astype(jnp.bfloat16)


def _topk_body(feat_ref, sq_ref, sqr_ref, idx_ref):
    b = pl.program_id(0)
    feat = feat_ref[...]
    sq = sq_ref[...]
    ctr = feat_ref[pl.ds(b * BLK, BLK), :]
    mm = lax.dot_general(_bf(feat), _bf(ctr), (((1,), (1,)), ((), ())),
                         preferred_element_type=jnp.float32)
    sqi_row = sqr_ref[:, pl.ds(b * BLK, BLK)]
    dmat = (sqi_row - 2.0 * mm) + sq
    jj = lax.broadcasted_iota(jnp.int32, (N, BLK), 0)
    ii = b * BLK + lax.broadcasted_iota(jnp.int32, (N, BLK), 1)
    dmat = jnp.where(jj == ii, jnp.inf, dmat)
    rows = []
    for _ in range(K):
        m = jnp.min(dmat, axis=0, keepdims=True)
        cand = jnp.where(dmat == m, jj, N)
        sel = jnp.min(cand, axis=0, keepdims=True)
        rows.append(sel)
        dmat = jnp.where(jj == sel, jnp.inf, dmat)
    idx_ref[...] = jnp.concatenate(rows, axis=0)


def _topk(feat, sq):
    c = feat.shape[1]
    return pl.pallas_call(
        _topk_body,
        grid=(NBLK,),
        in_specs=[pl.BlockSpec((N, c), lambda b: (0, 0)),
                  pl.BlockSpec((N, 1), lambda b: (0, 0)),
                  pl.BlockSpec((1, N), lambda b: (0, 0))],
        out_specs=pl.BlockSpec((K, BLK), lambda b: (0, b)),
        out_shape=jax.ShapeDtypeStruct((K, N), jnp.int32),
    )(feat, sq[:, None], sq[None, :])


def _gather_rows(table, idx_flat):
    cp = table.shape[1]
    nw = 32
    b_per_w = (K * N) // nw
    ch = 128
    nch = b_per_w // ch
    idx2d = idx_flat.reshape(K * N // ch, ch)
    mesh = plsc.VectorSubcoreMesh(core_axis_name="c", subcore_axis_name="s")

    @functools.partial(
        pl.kernel, mesh=mesh,
        compiler_params=pltpu.CompilerParams(use_tc_tiling_on_sc=False),
        out_type=jax.ShapeDtypeStruct((K * N, cp), jnp.float32),
        scratch_types=[
            pltpu.VMEM((nch, ch), jnp.int32),
            pltpu.VMEM((ch, cp), jnp.float32),
            pltpu.VMEM((ch, cp), jnp.float32),
            pltpu.SemaphoreType.DMA,
            pltpu.SemaphoreType.DMA,
        ],
    )
    def gk(table_hbm, idx_hbm, out_hbm, idx_v, rows_a, rows_b, sem_a, sem_b):
        wid = lax.axis_index("s") * 2 + lax.axis_index("c")
        base = wid * b_per_w
        pltpu.sync_copy(idx_hbm.at[pl.ds(wid * nch, nch)], idx_v)
        bufs = (rows_a, rows_b)
        sems = (sem_a, sem_b)
        copies = [None] * nch
        copies[0] = pltpu.async_copy(table_hbm.at[idx_v.at[0]], bufs[0],
                                     sems[0])
        for c in range(nch):
            if c + 1 < nch:
                copies[c + 1] = pltpu.async_copy(
                    table_hbm.at[idx_v.at[c + 1]], bufs[(c + 1) % 2],
                    sems[(c + 1) % 2])
            copies[c].wait()
            pltpu.sync_copy(bufs[c % 2],
                            out_hbm.at[pl.ds(base + c * ch, ch)])

    return gk(table, idx2d)


def _mlp_body(nb_ref, feat_ref, w1_ref, b1_ref, w2_ref, b2_ref, w3_ref,
              b3_ref, out_ref, *, c, h):
    ctr = feat_ref[...]
    nbb = nb_ref[...]
    if nbb.shape[-1] != c:
        nbb = nbb[:, :, :c]
    ctrb = jnp.broadcast_to(ctr[None, :, :], (K, BLK, c))
    h0 = jnp.concatenate([ctrb, nbb - ctrb], axis=-1)
    hh = h0.reshape(K * BLK, 2 * c)
    hh = jnp.dot(_bf(hh), _bf(w1_ref[...]),
                 preferred_element_type=jnp.float32) + b1_ref[...]
    hh = jnp.maximum(hh, 0.0)
    hh = jnp.dot(_bf(hh), _bf(w2_ref[...]),
                 preferred_element_type=jnp.float32) + b2_ref[...]
    hh = jnp.maximum(hh, 0.0)
    hh = jnp.dot(_bf(hh), _bf(w3_ref[...]),
                 preferred_element_type=jnp.float32) + b3_ref[...]
    h3 = hh.reshape(K, BLK, h)
    acc = h3[0]
    for k in range(1, K):
        acc = jnp.maximum(acc, h3[k])
    out_ref[...] = acc


def _mlp(nb3, feat, ws, bs):
    c = feat.shape[1]
    cp = nb3.shape[2]
    h = ws[2].shape[1]
    return pl.pallas_call(
        functools.partial(_mlp_body, c=c, h=h),
        grid=(NBLK,),
        in_specs=[
            pl.BlockSpec((K, BLK, cp), lambda b: (0, b, 0)),
            pl.BlockSpec((BLK, c), lambda b: (b, 0)),
            pl.BlockSpec(ws[0].shape, lambda b: (0, 0)),
            pl.BlockSpec(bs[0].shape, lambda b: (0,)),
            pl.BlockSpec(ws[1].shape, lambda b: (0, 0)),
            pl.BlockSpec(bs[1].shape, lambda b: (0,)),
            pl.BlockSpec(ws[2].shape, lambda b: (0, 0)),
            pl.BlockSpec(bs[2].shape, lambda b: (0,)),
        ],
        out_specs=pl.BlockSpec((BLK, h), lambda b: (b, 0)),
        out_shape=jax.ShapeDtypeStruct((N, h), jnp.float32),
    )(nb3, feat, ws[0], bs[0], ws[1], bs[1], ws[2], bs[2])


def kernel(x, params):
    feat = x
    for ws, bs in params:
        c = feat.shape[1]
        sq = jnp.sum(feat * feat, axis=1)
        idx = _topk(feat, sq)
        if c % 16:
            table = jnp.pad(feat, ((0, 0), (0, 16 - c)))
        else:
            table = feat
        nb = _gather_rows(table, idx.reshape(-1))
        nb3 = nb.reshape(K, N, table.shape[1])
        feat = _mlp(nb3, feat, ws, bs)
    return feat


# speedup vs baseline: 13.1159x; 1.0742x over previous
import functools

import jax
import jax.numpy as jnp
from jax import lax
from jax.experimental import pallas as pl
from jax.experimental.pallas import tpu as pltpu
from jax.experimental.pallas import tpu_sc as plsc

N = 2048
K = 16
BLK = 256
TBLK = 512
NBLK = N // BLK


def _bf(a):
    return a.astype(jnp.bfloat16)


def _topk_body(feat_ref, sq_ref, sqr_ref, idx_ref):
    b = pl.program_id(0)
    feat = feat_ref[...]
    sq = sq_ref[...]
    ctr = feat_ref[pl.ds(b * TBLK, TBLK), :]
    mm = lax.dot_general(_bf(feat), _bf(ctr), (((1,), (1,)), ((), ())),
                         preferred_element_type=jnp.float32)
    sqi_row = sqr_ref[:, pl.ds(b * TBLK, TBLK)]
    dmat = (sqi_row - 2.0 * mm) + sq
    jj = lax.broadcasted_iota(jnp.int32, (N, TBLK), 0)
    ii = b * TBLK + lax.broadcasted_iota(jnp.int32, (N, TBLK), 1)
    dmat = jnp.where(jj == ii, jnp.inf, dmat)
    rows = []
    for _ in range(K):
        m = jnp.min(dmat, axis=0, keepdims=True)
        cand = jnp.where(dmat == m, jj, N)
        sel = jnp.min(cand, axis=0, keepdims=True)
        rows.append(sel)
        dmat = jnp.where(jj == sel, jnp.inf, dmat)
    idx_ref[...] = jnp.concatenate(rows, axis=0)


def _topk(feat, sq):
    c = feat.shape[1]
    return pl.pallas_call(
        _topk_body,
        grid=(N // TBLK,),
        in_specs=[pl.BlockSpec((N, c), lambda b: (0, 0)),
                  pl.BlockSpec((N, 1), lambda b: (0, 0)),
                  pl.BlockSpec((1, N), lambda b: (0, 0))],
        out_specs=pl.BlockSpec((K, TBLK), lambda b: (0, b)),
        out_shape=jax.ShapeDtypeStruct((K, N), jnp.int32),
    )(feat, sq[:, None], sq[None, :])


def _gather_rows(table, idx_flat):
    cp = table.shape[1]
    nw = 32
    b_per_w = (K * N) // nw
    ch = 128
    nch = b_per_w // ch
    idx2d = idx_flat.reshape(K * N // ch, ch)
    mesh = plsc.VectorSubcoreMesh(core_axis_name="c", subcore_axis_name="s")

    @functools.partial(
        pl.kernel, mesh=mesh,
        compiler_params=pltpu.CompilerParams(use_tc_tiling_on_sc=False),
        out_type=jax.ShapeDtypeStruct((K * N, cp), jnp.float32),
        scratch_types=[
            pltpu.VMEM((nch, ch), jnp.int32),
            pltpu.VMEM((ch, cp), jnp.float32),
            pltpu.VMEM((ch, cp), jnp.float32),
            pltpu.SemaphoreType.DMA,
            pltpu.SemaphoreType.DMA,
        ],
    )
    def gk(table_hbm, idx_hbm, out_hbm, idx_v, rows_a, rows_b, sem_a, sem_b):
        wid = lax.axis_index("s") * 2 + lax.axis_index("c")
        base = wid * b_per_w
        pltpu.sync_copy(idx_hbm.at[pl.ds(wid * nch, nch)], idx_v)
        bufs = (rows_a, rows_b)
        sems = (sem_a, sem_b)
        copies = [None] * nch
        copies[0] = pltpu.async_copy(table_hbm.at[idx_v.at[0]], bufs[0],
                                     sems[0])
        for c in range(nch):
            if c + 1 < nch:
                copies[c + 1] = pltpu.async_copy(
                    table_hbm.at[idx_v.at[c + 1]], bufs[(c + 1) % 2],
                    sems[(c + 1) % 2])
            copies[c].wait()
            pltpu.sync_copy(bufs[c % 2],
                            out_hbm.at[pl.ds(base + c * ch, ch)])

    return gk(table, idx2d)


def _mlp_body(nb_ref, feat_ref, w1_ref, b1_ref, w2_ref, b2_ref, w3_ref,
              b3_ref, out_ref, *, c, h):
    ctr = feat_ref[...]
    nbb = nb_ref[...]
    if nbb.shape[-1] != c:
        nbb = nbb[:, :, :c]
    ctrb = jnp.broadcast_to(ctr[None, :, :], (K, BLK, c))
    h0 = jnp.concatenate([ctrb, nbb - ctrb], axis=-1)
    hh = h0.reshape(K * BLK, 2 * c)
    hh = jnp.dot(_bf(hh), _bf(w1_ref[...]),
                 preferred_element_type=jnp.float32) + b1_ref[...]
    hh = jnp.maximum(hh, 0.0)
    hh = jnp.dot(_bf(hh), _bf(w2_ref[...]),
                 preferred_element_type=jnp.float32) + b2_ref[...]
    hh = jnp.maximum(hh, 0.0)
    hh = jnp.dot(_bf(hh), _bf(w3_ref[...]),
                 preferred_element_type=jnp.float32) + b3_ref[...]
    h3 = hh.reshape(K, BLK, h)
    acc = h3[0]
    for k in range(1, K):
        acc = jnp.maximum(acc, h3[k])
    out_ref[...] = acc


def _mlp(nb3, feat, ws, bs):
    c = feat.shape[1]
    cp = nb3.shape[2]
    h = ws[2].shape[1]
    return pl.pallas_call(
        functools.partial(_mlp_body, c=c, h=h),
        grid=(NBLK,),
        in_specs=[
            pl.BlockSpec((K, BLK, cp), lambda b: (0, b, 0)),
            pl.BlockSpec((BLK, c), lambda b: (b, 0)),
            pl.BlockSpec(ws[0].shape, lambda b: (0, 0)),
            pl.BlockSpec(bs[0].shape, lambda b: (0,)),
            pl.BlockSpec(ws[1].shape, lambda b: (0, 0)),
            pl.BlockSpec(bs[1].shape, lambda b: (0,)),
            pl.BlockSpec(ws[2].shape, lambda b: (0, 0)),
            pl.BlockSpec(bs[2].shape, lambda b: (0,)),
        ],
        out_specs=pl.BlockSpec((BLK, h), lambda b: (b, 0)),
        out_shape=jax.ShapeDtypeStruct((N, h), jnp.float32),
    )(nb3, feat, ws[0], bs[0], ws[1], bs[1], ws[2], bs[2])


def kernel(x, params):
    feat = x
    for ws, bs in params:
        c = feat.shape[1]
        sq = jnp.sum(feat * feat, axis=1)
        idx = _topk(feat, sq)
        if c % 16:
            table = jnp.pad(feat, ((0, 0), (0, 16 - c)))
        else:
            table = feat
        nb = _gather_rows(table, idx.reshape(-1))
        nb3 = nb.reshape(K, N, table.shape[1])
        feat = _mlp(nb3, feat, ws, bs)
    return feat
